# Initial kernel scaffold; baseline (speedup 1.0000x reference)
import jax
import jax.numpy as jnp
from jax.experimental import pallas as pl


def kernel(sample_points, closest_points, bound, grid_size, planes, axes):
    raise NotImplementedError("write your pallas kernel here")


# speedup vs baseline: 39.3707x; 39.3707x over previous
import jax
import jax.numpy as jnp
from jax import lax
from jax.experimental import pallas as pl
from jax.experimental.pallas import tpu as pltpu
from jax.experimental.pallas import tpu_sc as plsc

G = 32
NPB = 32768
CHUNK = 2048
NCHUNK = NPB // CHUNK
GROUPS = CHUNK // 16


def _sc_body(pts_hbm, grid_hbm, coef_hbm, partials_hbm,
             grid_v, pts_v, coef_v, acc_v):
    h = lax.axis_index("c")
    b = lax.axis_index("s")

    pltpu.sync_copy(coef_hbm, coef_v)
    pltpu.sync_copy(
        grid_hbm.at[pl.ds(b * (G * G * G * 3), G * G * G * 3)], grid_v)

    zero16 = jnp.zeros((16,), jnp.float32)
    for j in range(24):
        acc_v[j] = zero16

    v1 = coef_v[b, pl.ds(0, 16)]
    v2 = coef_v[b, pl.ds(16, 16)]

    iota3 = lax.iota(jnp.int32, 16) * 3
    goffv = jnp.full((16,), v1[15], jnp.float32) * jnp.float32(G)
    gmax = jnp.full((16,), jnp.float32(G - 1), jnp.float32)
    gzero = jnp.zeros((16,), jnp.float32)

    def cell_and_acc(slot, px, py, pz):
        fx = jnp.minimum(jnp.maximum(px * jnp.float32(G) + goffv, gzero), gmax)
        fy = jnp.minimum(jnp.maximum(py * jnp.float32(G) + goffv, gzero), gmax)
        fz = jnp.minimum(jnp.maximum(pz * jnp.float32(G) + goffv, gzero), gmax)
        lin3 = (fx.astype(jnp.int32) * (G * G)
                + fy.astype(jnp.int32) * G + fz.astype(jnp.int32)) * 3
        cx = plsc.load_gather(grid_v, [lin3])
        cy = plsc.load_gather(grid_v, [lin3 + 1])
        cz = plsc.load_gather(grid_v, [lin3 + 2])
        dx = px - cx
        dy = py - cy
        dz = pz - cz
        plsc.addupdate(acc_v.at[3 * slot + 0], dx * dx)
        plsc.addupdate(acc_v.at[3 * slot + 1], dy * dy)
        plsc.addupdate(acc_v.at[3 * slot + 2], dz * dz)

    def chunk_body(k, carry):
        base = b * (2 * NPB * 3) + h * (NPB * 3) + k * (CHUNK * 3)
        pltpu.sync_copy(pts_hbm.at[pl.ds(base, CHUNK * 3)], pts_v)

        for t in range(3):
            nxv = jnp.full((16,), v1[4 * t], jnp.float32)
            nyv = jnp.full((16,), v1[4 * t + 1], jnp.float32)
            nzv = jnp.full((16,), v1[4 * t + 2], jnp.float32)
            dv = jnp.full((16,), v1[4 * t + 3], jnp.float32)
            s2v = jnp.float32(2.0) / (nxv * nxv + nyv * nyv + nzv * nzv)
            uxv = s2v * nxv
            uyv = s2v * nyv
            uzv = s2v * nzv
            wxv = dv * uxv
            wyv = dv * uyv
            wzv = dv * uzv

            def grp_refl(g2, c, t=t, nxv=nxv, nyv=nyv, nzv=nzv, uxv=uxv,
                         uyv=uyv, uzv=uzv, wxv=wxv, wyv=wyv, wzv=wzv):
                i0 = iota3 + g2 * 48
                x = plsc.load_gather(pts_v, [i0])
                y = plsc.load_gather(pts_v, [i0 + 1])
                z = plsc.load_gather(pts_v, [i0 + 2])
                dot = x * nxv + y * nyv + z * nzv
                px = x - dot * uxv - wxv
                py = y - dot * uyv - wyv
                pz = z - dot * uzv - wzv
                cell_and_acc(t, px, py, pz)
                return c

            lax.fori_loop(0, GROUPS, grp_refl, 0)

        for t in range(3):
            if t == 0:
                q1, q2, q3 = v1[12], v1[13], v1[14]
            else:
                q1, q2, q3 = v2[3 * t - 3], v2[3 * t - 2], v2[3 * t - 1]
            q1v = jnp.full((16,), q1, jnp.float32)
            q2v = jnp.full((16,), q2, jnp.float32)
            q3v = jnp.full((16,), q3, jnp.float32)
            sxv = -(q1v * q1v)
            syv = -(q2v * q2v)
            szv = -(q3v * q3v)

            def grp_rot(g2, c, t=t, sxv=sxv, syv=syv, szv=szv):
                i0 = iota3 + g2 * 48
                x = plsc.load_gather(pts_v, [i0])
                y = plsc.load_gather(pts_v, [i0 + 1])
                z = plsc.load_gather(pts_v, [i0 + 2])
                cell_and_acc(3 + t, x * sxv, y * syv, z * szv)
                return c

            lax.fori_loop(0, GROUPS, grp_rot, 0)
        return carry

    lax.fori_loop(0, NCHUNK, chunk_body, 0)

    pltpu.sync_copy(acc_v, partials_hbm.at[h, b])


def _finish_body(p_ref, o_ref):
    p = p_ref[...]
    s = jnp.sum(p, axis=(0, 3))
    o_ref[0, 0] = jnp.sum(jnp.sqrt(s)) * jnp.float32(1.0 / 3.0)


def kernel(sample_points, closest_points, bound, grid_size, planes, axes):
    del grid_size
    B, N, _ = sample_points.shape
    pts_flat = sample_points.reshape(B * N * 3)

    pr = jnp.transpose(planes, (1, 0, 2)).reshape(B, 12)
    ar = jnp.transpose(axes[:, :, 1:4], (1, 0, 2)).reshape(B, 9)
    bb = jnp.broadcast_to(bound.reshape(1, 1), (B, 1))
    coef = jnp.concatenate(
        [pr, ar[:, 0:3], bb, ar[:, 3:9], jnp.zeros((B, 10), jnp.float32)],
        axis=1)

    mesh = plsc.VectorSubcoreMesh(core_axis_name="c", subcore_axis_name="s")
    sc = pl.kernel(
        _sc_body,
        out_type=jax.ShapeDtypeStruct((2, 16, 24, 16), jnp.float32),
        mesh=mesh,
        scratch_types=[
            pltpu.VMEM((G * G * G * 3,), jnp.float32),
            pltpu.VMEM((CHUNK * 3,), jnp.float32),
            pltpu.VMEM((16, 32), jnp.float32),
            pltpu.VMEM((24, 16), jnp.float32),
        ],
        compiler_params=pltpu.CompilerParams(
            needs_layout_passes=False, use_tc_tiling_on_sc=False),
    )
    partials = sc(pts_flat, closest_points.reshape(-1), coef)

    out = pl.pallas_call(
        _finish_body,
        out_shape=jax.ShapeDtypeStruct((1, 1), jnp.float32),
        out_specs=pl.BlockSpec(memory_space=pltpu.SMEM),
    )(partials)
    return out.reshape(1)


# speedup vs baseline: 41.2020x; 1.0465x over previous
import jax
import jax.numpy as jnp
from jax import lax
from jax.experimental import pallas as pl
from jax.experimental.pallas import tpu as pltpu
from jax.experimental.pallas import tpu_sc as plsc

G = 32
NPB = 32768
CHUNK = 2048
NCHUNK = NPB // CHUNK
GROUPS = CHUNK // 16


def _sc_body(pts_hbm, grid_hbm, coef_hbm, partials_hbm,
             grid_v, pts_v, coef_v, acc_v):
    h = lax.axis_index("c")
    b = lax.axis_index("s")

    pltpu.sync_copy(coef_hbm, coef_v)
    pltpu.sync_copy(
        grid_hbm.at[pl.ds(b * (G * G * G * 3), G * G * G * 3)], grid_v)

    zero16 = jnp.zeros((16,), jnp.float32)
    for j in range(24):
        acc_v[j] = zero16

    v1 = coef_v[b, pl.ds(0, 16)]
    v2 = coef_v[b, pl.ds(16, 16)]

    iota3 = lax.iota(jnp.int32, 16) * 3
    goffv = jnp.full((16,), v1[15], jnp.float32) * jnp.float32(G)
    gmax = jnp.full((16,), jnp.float32(G - 1), jnp.float32)
    gzero = jnp.zeros((16,), jnp.float32)

    def cell_sq(px, py, pz):
        fx = jnp.minimum(jnp.maximum(px * jnp.float32(G) + goffv, gzero), gmax)
        fy = jnp.minimum(jnp.maximum(py * jnp.float32(G) + goffv, gzero), gmax)
        fz = jnp.minimum(jnp.maximum(pz * jnp.float32(G) + goffv, gzero), gmax)
        lin3 = (fx.astype(jnp.int32) * (G * G)
                + fy.astype(jnp.int32) * G + fz.astype(jnp.int32)) * 3
        cx = plsc.load_gather(grid_v, [lin3])
        cy = plsc.load_gather(grid_v, [lin3 + 1])
        cz = plsc.load_gather(grid_v, [lin3 + 2])
        dx = px - cx
        dy = py - cy
        dz = pz - cz
        return dx * dx, dy * dy, dz * dz

    def acc_flush(slot, a0, a1, a2):
        acc_v[3 * slot + 0] = acc_v[3 * slot + 0] + a0
        acc_v[3 * slot + 1] = acc_v[3 * slot + 1] + a1
        acc_v[3 * slot + 2] = acc_v[3 * slot + 2] + a2

    def chunk_body(k, carry):
        base = b * (2 * NPB * 3) + h * (NPB * 3) + k * (CHUNK * 3)
        pltpu.sync_copy(pts_hbm.at[pl.ds(base, CHUNK * 3)], pts_v)

        for t in range(3):
            nxv = jnp.full((16,), v1[4 * t], jnp.float32)
            nyv = jnp.full((16,), v1[4 * t + 1], jnp.float32)
            nzv = jnp.full((16,), v1[4 * t + 2], jnp.float32)
            dv = jnp.full((16,), v1[4 * t + 3], jnp.float32)
            s2v = jnp.float32(2.0) / (nxv * nxv + nyv * nyv + nzv * nzv)
            uxv = s2v * nxv
            uyv = s2v * nyv
            uzv = s2v * nzv
            wxv = dv * uxv
            wyv = dv * uyv
            wzv = dv * uzv

            def grp_refl(g2, acc, nxv=nxv, nyv=nyv, nzv=nzv, uxv=uxv,
                         uyv=uyv, uzv=uzv, wxv=wxv, wyv=wyv, wzv=wzv):
                a0, a1, a2 = acc
                i0 = iota3 + g2 * 48
                x = plsc.load_gather(pts_v, [i0])
                y = plsc.load_gather(pts_v, [i0 + 1])
                z = plsc.load_gather(pts_v, [i0 + 2])
                dot = x * nxv + y * nyv + z * nzv
                px = x - dot * uxv - wxv
                py = y - dot * uyv - wyv
                pz = z - dot * uzv - wzv
                s0, s1, s2 = cell_sq(px, py, pz)
                return a0 + s0, a1 + s1, a2 + s2

            a0, a1, a2 = plsc.parallel_loop(
                0, GROUPS, carry=(zero16, zero16, zero16), unroll=4)(grp_refl)
            acc_flush(t, a0, a1, a2)

        for t in range(3):
            if t == 0:
                q1, q2, q3 = v1[12], v1[13], v1[14]
            else:
                q1, q2, q3 = v2[3 * t - 3], v2[3 * t - 2], v2[3 * t - 1]
            q1v = jnp.full((16,), q1, jnp.float32)
            q2v = jnp.full((16,), q2, jnp.float32)
            q3v = jnp.full((16,), q3, jnp.float32)
            sxv = -(q1v * q1v)
            syv = -(q2v * q2v)
            szv = -(q3v * q3v)

            def grp_rot(g2, acc, sxv=sxv, syv=syv, szv=szv):
                a0, a1, a2 = acc
                i0 = iota3 + g2 * 48
                x = plsc.load_gather(pts_v, [i0])
                y = plsc.load_gather(pts_v, [i0 + 1])
                z = plsc.load_gather(pts_v, [i0 + 2])
                s0, s1, s2 = cell_sq(x * sxv, y * syv, z * szv)
                return a0 + s0, a1 + s1, a2 + s2

            a0, a1, a2 = plsc.parallel_loop(
                0, GROUPS, carry=(zero16, zero16, zero16), unroll=4)(grp_rot)
            acc_flush(3 + t, a0, a1, a2)
        return carry

    lax.fori_loop(0, NCHUNK, chunk_body, 0)

    pltpu.sync_copy(acc_v, partials_hbm.at[h, b])


def _finish_body(p_ref, o_ref):
    p = p_ref[...]
    s = jnp.sum(p, axis=(0, 3))
    o_ref[0, 0] = jnp.sum(jnp.sqrt(s)) * jnp.float32(1.0 / 3.0)


def kernel(sample_points, closest_points, bound, grid_size, planes, axes):
    del grid_size
    B, N, _ = sample_points.shape
    pts_flat = sample_points.reshape(B * N * 3)

    pr = jnp.transpose(planes, (1, 0, 2)).reshape(B, 12)
    ar = jnp.transpose(axes[:, :, 1:4], (1, 0, 2)).reshape(B, 9)
    bb = jnp.broadcast_to(bound.reshape(1, 1), (B, 1))
    coef = jnp.concatenate(
        [pr, ar[:, 0:3], bb, ar[:, 3:9], jnp.zeros((B, 10), jnp.float32)],
        axis=1)

    mesh = plsc.VectorSubcoreMesh(core_axis_name="c", subcore_axis_name="s")
    sc = pl.kernel(
        _sc_body,
        out_type=jax.ShapeDtypeStruct((2, 16, 24, 16), jnp.float32),
        mesh=mesh,
        scratch_types=[
            pltpu.VMEM((G * G * G * 3,), jnp.float32),
            pltpu.VMEM((CHUNK * 3,), jnp.float32),
            pltpu.VMEM((16, 32), jnp.float32),
            pltpu.VMEM((24, 16), jnp.float32),
        ],
        compiler_params=pltpu.CompilerParams(
            needs_layout_passes=False, use_tc_tiling_on_sc=False),
    )
    partials = sc(pts_flat, closest_points.reshape(-1), coef)

    out = pl.pallas_call(
        _finish_body,
        out_shape=jax.ShapeDtypeStruct((1, 1), jnp.float32),
        out_specs=pl.BlockSpec(memory_space=pltpu.SMEM),
    )(partials)
    return out.reshape(1)


# speedup vs baseline: 520.6035x; 12.6354x over previous
import jax
import jax.numpy as jnp
from jax import lax
from jax.experimental import pallas as pl
from jax.experimental.pallas import tpu as pltpu
from jax.experimental.pallas import tpu_sc as plsc

G = 32
GG = G * G * G
NPB = 32768
CHUNK = 4096
NCHUNK = NPB // CHUNK
GROUPS = CHUNK // 16
N = 65536
B = 16


def _sc_body(pts_hbm, grid_hbm, coef_hbm, partials_hbm,
             gx_v, gy_v, gz_v, pbuf_v, coef_v, acc_v, sem0, sem1):
    h = lax.axis_index("c")
    b = lax.axis_index("s")

    def chunk_copy(k, slot, sem):
        base = h * NPB + k * CHUNK
        return pltpu.make_async_copy(
            pts_hbm.at[:, b, pl.ds(base, CHUNK)], pbuf_v.at[slot], sem)

    chunk_copy(0, 0, sem0).start()

    pltpu.sync_copy(coef_hbm, coef_v)
    pltpu.sync_copy(grid_hbm.at[0, pl.ds(b * GG, GG)], gx_v)
    pltpu.sync_copy(grid_hbm.at[1, pl.ds(b * GG, GG)], gy_v)
    pltpu.sync_copy(grid_hbm.at[2, pl.ds(b * GG, GG)], gz_v)

    zero16 = jnp.zeros((16,), jnp.float32)
    for j in range(24):
        acc_v[j] = zero16

    v1 = coef_v[b, pl.ds(0, 16)]
    v2 = coef_v[b, pl.ds(16, 16)]

    goffv = jnp.full((16,), v1[15], jnp.float32) * jnp.float32(G)
    gmax = jnp.full((16,), jnp.float32(G - 1), jnp.float32)
    gzero = jnp.zeros((16,), jnp.float32)

    def cell_sq(px, py, pz):
        fx = jnp.minimum(jnp.maximum(px * jnp.float32(G) + goffv, gzero), gmax)
        fy = jnp.minimum(jnp.maximum(py * jnp.float32(G) + goffv, gzero), gmax)
        fz = jnp.minimum(jnp.maximum(pz * jnp.float32(G) + goffv, gzero), gmax)
        lin = (fx.astype(jnp.int32) * (G * G)
               + fy.astype(jnp.int32) * G + fz.astype(jnp.int32))
        dx = px - plsc.load_gather(gx_v, [lin])
        dy = py - plsc.load_gather(gy_v, [lin])
        dz = pz - plsc.load_gather(gz_v, [lin])
        return dx * dx, dy * dy, dz * dz

    def acc_flush(slot, a0, a1, a2):
        acc_v[3 * slot + 0] = acc_v[3 * slot + 0] + a0
        acc_v[3 * slot + 1] = acc_v[3 * slot + 1] + a1
        acc_v[3 * slot + 2] = acc_v[3 * slot + 2] + a2

    def compute_chunk(slot):
        for t in range(3):
            nxv = jnp.full((16,), v1[4 * t], jnp.float32)
            nyv = jnp.full((16,), v1[4 * t + 1], jnp.float32)
            nzv = jnp.full((16,), v1[4 * t + 2], jnp.float32)
            dv = jnp.full((16,), v1[4 * t + 3], jnp.float32)
            s2v = jnp.float32(2.0) / (nxv * nxv + nyv * nyv + nzv * nzv)
            uxv = s2v * nxv
            uyv = s2v * nyv
            uzv = s2v * nzv
            wxv = dv * uxv
            wyv = dv * uyv
            wzv = dv * uzv

            def grp_refl(g2, acc, nxv=nxv, nyv=nyv, nzv=nzv, uxv=uxv,
                         uyv=uyv, uzv=uzv, wxv=wxv, wyv=wyv, wzv=wzv):
                a0, a1, a2 = acc
                x = pbuf_v[slot, 0, pl.ds(g2 * 16, 16)]
                y = pbuf_v[slot, 1, pl.ds(g2 * 16, 16)]
                z = pbuf_v[slot, 2, pl.ds(g2 * 16, 16)]
                dot = x * nxv + y * nyv + z * nzv
                px = x - dot * uxv - wxv
                py = y - dot * uyv - wyv
                pz = z - dot * uzv - wzv
                s0, s1, s2 = cell_sq(px, py, pz)
                return a0 + s0, a1 + s1, a2 + s2

            a0, a1, a2 = plsc.parallel_loop(
                0, GROUPS, carry=(zero16, zero16, zero16), unroll=4)(grp_refl)
            acc_flush(t, a0, a1, a2)

        for t in range(3):
            if t == 0:
                q1, q2, q3 = v1[12], v1[13], v1[14]
            else:
                q1, q2, q3 = v2[3 * t - 3], v2[3 * t - 2], v2[3 * t - 1]
            q1v = jnp.full((16,), q1, jnp.float32)
            q2v = jnp.full((16,), q2, jnp.float32)
            q3v = jnp.full((16,), q3, jnp.float32)
            sxv = -(q1v * q1v)
            syv = -(q2v * q2v)
            szv = -(q3v * q3v)

            def grp_rot(g2, acc, sxv=sxv, syv=syv, szv=szv):
                a0, a1, a2 = acc
                x = pbuf_v[slot, 0, pl.ds(g2 * 16, 16)]
                y = pbuf_v[slot, 1, pl.ds(g2 * 16, 16)]
                z = pbuf_v[slot, 2, pl.ds(g2 * 16, 16)]
                s0, s1, s2 = cell_sq(x * sxv, y * syv, z * szv)
                return a0 + s0, a1 + s1, a2 + s2

            a0, a1, a2 = plsc.parallel_loop(
                0, GROUPS, carry=(zero16, zero16, zero16), unroll=4)(grp_rot)
            acc_flush(3 + t, a0, a1, a2)

    def pair_body(kk, carry):
        k0 = 2 * kk
        chunk_copy(k0, 0, sem0).wait()
        chunk_copy(k0 + 1, 1, sem1).start()
        compute_chunk(0)
        chunk_copy(k0 + 1, 1, sem1).wait()

        @pl.when(kk < NCHUNK // 2 - 1)
        def _():
            chunk_copy(k0 + 2, 0, sem0).start()

        compute_chunk(1)
        return carry

    lax.fori_loop(0, NCHUNK // 2, pair_body, 0)

    pltpu.sync_copy(acc_v, partials_hbm.at[h, b])


def _finish_body(p_ref, o_ref):
    p = p_ref[...]
    s = jnp.sum(p, axis=(0, 3))
    o_ref[0, 0] = jnp.sum(jnp.sqrt(s)) * jnp.float32(1.0 / 3.0)


def kernel(sample_points, closest_points, bound, grid_size, planes, axes):
    del grid_size
    pts_soa = jnp.transpose(sample_points, (2, 0, 1))
    grid_soa = jnp.transpose(closest_points, (1, 0))

    pr = jnp.transpose(planes, (1, 0, 2)).reshape(B, 12)
    ar = jnp.transpose(axes[:, :, 1:4], (1, 0, 2)).reshape(B, 9)
    bb = jnp.broadcast_to(bound.reshape(1, 1), (B, 1))
    coef = jnp.concatenate(
        [pr, ar[:, 0:3], bb, ar[:, 3:9], jnp.zeros((B, 10), jnp.float32)],
        axis=1)

    mesh = plsc.VectorSubcoreMesh(core_axis_name="c", subcore_axis_name="s")
    sc = pl.kernel(
        _sc_body,
        out_type=jax.ShapeDtypeStruct((2, 16, 24, 16), jnp.float32),
        mesh=mesh,
        scratch_types=[
            pltpu.VMEM((GG,), jnp.float32),
            pltpu.VMEM((GG,), jnp.float32),
            pltpu.VMEM((GG,), jnp.float32),
            pltpu.VMEM((2, 3, CHUNK), jnp.float32),
            pltpu.VMEM((16, 32), jnp.float32),
            pltpu.VMEM((24, 16), jnp.float32),
            pltpu.SemaphoreType.DMA,
            pltpu.SemaphoreType.DMA,
        ],
        compiler_params=pltpu.CompilerParams(
            needs_layout_passes=False, use_tc_tiling_on_sc=False),
    )
    partials = sc(pts_soa, grid_soa, coef)

    out = pl.pallas_call(
        _finish_body,
        out_shape=jax.ShapeDtypeStruct((1, 1), jnp.float32),
        out_specs=pl.BlockSpec(memory_space=pltpu.SMEM),
    )(partials)
    return out.reshape(1)
